# Initial kernel scaffold; baseline (speedup 1.0000x reference)
import jax
import jax.numpy as jnp
from jax.experimental import pallas as pl


def kernel(x, fc_weight, fc_bias, ffm_tables):
    raise NotImplementedError("write your pallas kernel here")


# speedup vs baseline: 13.4273x; 13.4273x over previous
import functools

import numpy as np
import jax
import jax.numpy as jnp
from jax import lax
from jax.experimental import pallas as pl
from jax.experimental.pallas import tpu as pltpu
from jax.experimental.pallas import tpu_sc as plsc

F = 26
D = 16
PER_FIELD = 3846
TOTAL = F * PER_FIELD
B = 16384

NC, NS = 2, 16
NW = NC * NS
BPW = B // NW
C = 4
NP = F * (F - 1) // 2
NPP = 328
ROWS = C * NPP
NCHUNK = BPW // C
SEG = 128

_PI, _PJ = np.triu_indices(F, 1)
_CA = (_PJ.astype(np.int64) * TOTAL + _PI * PER_FIELD).astype(np.int32)
_CB = (_PI.astype(np.int64) * TOTAL + _PJ * PER_FIELD).astype(np.int32)
PI_ARR = _PI.astype(np.int32)
PJ_ARR = _PJ.astype(np.int32)
FIELD_OFF = (np.arange(F) * PER_FIELD).astype(np.int32)


def _ffm_sc_body(tables, fc, idxa, idxb, idxf, biasv_h, out,
                 idxa_v, idxb_v, idxf_v, bufa, bufb, fcv, zbuf, bias_v, sems):
    cid = lax.axis_index("c")
    sid = lax.axis_index("s")
    wid = sid * NC + cid
    base = wid * BPW

    pltpu.sync_copy(biasv_h, bias_v)
    lanes = lax.iota(jnp.int32, 16)
    fcmask = lanes < (F - 16)
    lane0 = lanes == 0
    zero16 = jnp.zeros((16,), jnp.float32)
    nfull, rem = divmod(ROWS, SEG)

    def issue(ch, sl_buf):
        rowbase = (base + ch * C) * NPP
        ia = idxa_v.at[sl_buf]
        ib = idxb_v.at[sl_buf]
        iff = idxf_v.at[sl_buf]
        pltpu.sync_copy(idxa.at[pl.ds(rowbase, ROWS)], ia)
        pltpu.sync_copy(idxb.at[pl.ds(rowbase, ROWS)], ib)
        fbase = (base + ch * C) * 32
        pltpu.sync_copy(idxf.at[pl.ds(fbase, C * 32)], iff)
        sem = sems.at[sl_buf]
        for k in range(nfull):
            sl = pl.ds(k * SEG, SEG)
            pltpu.async_copy(tables.at[ia.at[sl]], bufa.at[sl_buf].at[sl], sem)
            pltpu.async_copy(tables.at[ib.at[sl]], bufb.at[sl_buf].at[sl], sem)
        if rem:
            sl = pl.ds(nfull * SEG, rem)
            pltpu.async_copy(tables.at[ia.at[sl]], bufa.at[sl_buf].at[sl], sem)
            pltpu.async_copy(tables.at[ib.at[sl]], bufb.at[sl_buf].at[sl], sem)
        pltpu.async_copy(fc.at[iff], fcv.at[sl_buf], sem)

    def waitall(sl_buf):
        sem = sems.at[sl_buf]
        ia = idxa_v.at[sl_buf]
        ib = idxb_v.at[sl_buf]
        iff = idxf_v.at[sl_buf]
        for k in range(nfull):
            sl = pl.ds(k * SEG, SEG)
            pltpu.make_async_copy(tables.at[ia.at[sl]], bufa.at[sl_buf].at[sl], sem).wait()
            pltpu.make_async_copy(tables.at[ib.at[sl]], bufb.at[sl_buf].at[sl], sem).wait()
        if rem:
            sl = pl.ds(nfull * SEG, rem)
            pltpu.make_async_copy(tables.at[ia.at[sl]], bufa.at[sl_buf].at[sl], sem).wait()
            pltpu.make_async_copy(tables.at[ib.at[sl]], bufb.at[sl_buf].at[sl], sem).wait()
        pltpu.make_async_copy(fc.at[iff], fcv.at[sl_buf], sem).wait()

    def compute(ch, sl_buf):
        ba = bufa.at[sl_buf]
        bb = bufb.at[sl_buf]
        fcb = fcv.at[sl_buf]
        for s in range(C):
            for q in range(NP, NPP):
                ba[s * NPP + q] = zero16
        for s in range(C):
            rb = s * NPP

            @plsc.parallel_loop(0, NPP, 4, unroll=2,
                                carry=(zero16, zero16, zero16, zero16))
            def accs(p, carry):
                a0, a1, a2, a3 = carry
                return (
                    a0 + ba[rb + p] * bb[rb + p],
                    a1 + ba[rb + p + 1] * bb[rb + p + 1],
                    a2 + ba[rb + p + 2] * bb[rb + p + 2],
                    a3 + ba[rb + p + 3] * bb[rb + p + 3],
                )

            acc = (accs[0] + accs[1]) + (accs[2] + accs[3])
            ffm = jnp.sum(acc)
            v1 = fcb[pl.ds(s * 32, 16)]
            v2 = fcb[pl.ds(s * 32 + 16, 16)]
            lin = jnp.sum(v1) + jnp.sum(jnp.where(fcmask, v2, 0.0))
            zval = ffm + lin
            pos = jnp.full((16,), ch * C + s, dtype=jnp.int32)
            plsc.store_scatter(zbuf, [pos], jnp.full((16,), zval), mask=lane0)

    issue(0, 0)

    def body2(g, carry):
        ch = 2 * g
        issue(ch + 1, 1)
        waitall(0)
        compute(ch, 0)

        @pl.when(ch + 2 < NCHUNK)
        def _():
            issue(ch + 2, 0)

        waitall(1)
        compute(ch + 1, 1)
        return carry

    lax.fori_loop(0, NCHUNK // 2, body2, 0)

    bias = bias_v[...]
    for v in range(BPW // 16):
        sl = pl.ds(v * 16, 16)
        z = zbuf[sl] + bias
        zbuf[sl] = 1.0 / (1.0 + jnp.exp(-z))
    pltpu.sync_copy(zbuf, out.at[pl.ds(base, BPW)])


@functools.lru_cache(maxsize=1)
def _get_ffm_sc():
    mesh = plsc.VectorSubcoreMesh(
        core_axis_name="c", subcore_axis_name="s", num_cores=NC, num_subcores=NS
    )
    return pl.kernel(
        _ffm_sc_body,
        out_type=jax.ShapeDtypeStruct((B,), jnp.float32),
        mesh=mesh,
        compiler_params=pltpu.CompilerParams(
            needs_layout_passes=False, use_tc_tiling_on_sc=False
        ),
        scratch_types=[
            pltpu.VMEM((2, ROWS), jnp.int32),
            pltpu.VMEM((2, ROWS), jnp.int32),
            pltpu.VMEM((2, C * 32), jnp.int32),
            pltpu.VMEM((2, ROWS, D), jnp.float32),
            pltpu.VMEM((2, ROWS, D), jnp.float32),
            pltpu.VMEM((2, C * 32), jnp.float32),
            pltpu.VMEM((BPW,), jnp.float32),
            pltpu.VMEM((16,), jnp.float32),
            pltpu.SemaphoreType.DMA((2,)),
        ],
    )


def kernel(x, fc_weight, fc_bias, ffm_tables):
    tables_flat = ffm_tables.reshape(F * TOTAL, D)
    fc_flat = fc_weight.reshape(TOTAL)
    idx_a = jnp.take(x, PI_ARR, axis=1) + jnp.asarray(_CA)[None, :]
    idx_b = jnp.take(x, PJ_ARR, axis=1) + jnp.asarray(_CB)[None, :]
    idx_a = jnp.pad(idx_a, ((0, 0), (0, NPP - NP))).reshape(-1)
    idx_b = jnp.pad(idx_b, ((0, 0), (0, NPP - NP))).reshape(-1)
    idx_f = jnp.pad(x + jnp.asarray(FIELD_OFF)[None, :], ((0, 0), (0, 32 - F))).reshape(-1)
    bias_vec = jnp.broadcast_to(fc_bias, (16,)).astype(jnp.float32)
    return _get_ffm_sc()(tables_flat, fc_flat, idx_a, idx_b, idx_f, bias_vec)
